# Initial kernel scaffold; baseline (speedup 1.0000x reference)
import jax
import jax.numpy as jnp
from jax.experimental import pallas as pl


def kernel(x, codebook):
    raise NotImplementedError("write your pallas kernel here")


# speedup vs baseline: 1.7324x; 1.7324x over previous
import functools

import jax
import jax.numpy as jnp
from jax import lax
from jax.experimental import pallas as pl
from jax.experimental.pallas import tpu as pltpu
from jax.experimental.pallas import tpu_sc as plsc

B, T, D = 64, 576, 64
K = 1024
TOK = B * T

BT = 512
G = TOK // BT

NC, NS = 2, 16
NW = NC * NS
BPW = TOK // NW
CH = 128
NCH = BPW // CH


KC = 128
NKC = K // KC


def _argmin_body(x_ref, cb_ref, idx_ref):
    x = x_ref[...]
    best = jnp.full((1, BT), jnp.inf, jnp.float32)
    besti = jnp.zeros((1, BT), jnp.int32)
    iota = lax.broadcasted_iota(jnp.int32, (KC, BT), 0)
    for c in range(NKC):
        cb_c = cb_ref[pl.ds(c * KC, KC), :]
        mm = lax.dot_general(cb_c, x, (((1,), (1,)), ((), ())),
                             preferred_element_type=jnp.float32)
        cbn = jnp.sum(cb_c * cb_c, axis=1)
        d = cbn[:, None] - 2.0 * mm
        m = jnp.min(d, axis=0, keepdims=True)
        i = jnp.min(jnp.where(d == m, iota + c * KC, K),
                    axis=0, keepdims=True)
        take = m < best
        besti = jnp.where(take, i, besti)
        best = jnp.minimum(best, m)
    idx_ref[...] = besti.reshape(1, 1, BT)


_tc_argmin = pl.pallas_call(
    _argmin_body,
    grid=(G,),
    in_specs=[
        pl.BlockSpec((BT, D), lambda i: (i, 0)),
        pl.BlockSpec((K, D), lambda i: (0, 0)),
    ],
    out_specs=pl.BlockSpec((1, 1, BT), lambda i: (i, 0, 0)),
    out_shape=jax.ShapeDtypeStruct((G, 1, BT), jnp.int32),
)


@functools.cache
def _make_sc_gather():
    @functools.partial(
        pl.kernel,
        out_type=jax.ShapeDtypeStruct((TOK, D), jnp.float32),
        mesh=plsc.VectorSubcoreMesh(core_axis_name="c", subcore_axis_name="s"),
        scratch_types=[
            pltpu.VMEM((NCH, CH), jnp.int32),
            pltpu.VMEM((BPW, D), jnp.float32),
            pltpu.SemaphoreType.DMA,
        ],
        compiler_params=pltpu.CompilerParams(use_tc_tiling_on_sc=False),
    )
    def _sc_gather(table_hbm, idx_hbm, out_hbm, idx_v, rows_v, sem):
        wid = lax.axis_index("s") * NC + lax.axis_index("c")
        pltpu.sync_copy(idx_hbm.at[wid], idx_v)
        cps = [
            pltpu.async_copy(table_hbm.at[idx_v.at[j]],
                             rows_v.at[pl.ds(j * CH, CH)], sem)
            for j in range(NCH)
        ]
        for cp in cps:
            cp.wait()
        pltpu.sync_copy(rows_v, out_hbm.at[pl.ds(wid * BPW, BPW)])

    return _sc_gather


def kernel(x, codebook):
    xf = x.reshape(TOK, D)
    idx = _tc_argmin(xf, codebook).reshape(NW, NCH, CH)
    q = _make_sc_gather()(codebook, idx)
    return q.reshape(B, T, D)


# speedup vs baseline: 1.9767x; 1.1411x over previous
import functools

import jax
import jax.numpy as jnp
from jax import lax
from jax.experimental import pallas as pl
from jax.experimental.pallas import tpu as pltpu
from jax.experimental.pallas import tpu_sc as plsc

B, T, D = 64, 576, 64
K = 1024
TOK = B * T

BT = 512
G = TOK // BT
KC = 128
NKC = K // KC

NC, NS = 2, 16
NW = NC * NS
BPW = TOK // NW
CH = 128
NCH = BPW // CH
NBUF = 3


def _argmin_body(x_ref, cb_ref, idx_ref):
    x = x_ref[...]
    best = jnp.full((1, BT), jnp.inf, jnp.float32)
    besti = jnp.zeros((1, BT), jnp.int32)
    iota = lax.broadcasted_iota(jnp.int32, (KC, BT), 0)
    for c in range(NKC):
        cb_c = cb_ref[pl.ds(c * KC, KC), :]
        mm = lax.dot_general(cb_c, x, (((1,), (1,)), ((), ())),
                             preferred_element_type=jnp.float32)
        cbn = jnp.sum(cb_c * cb_c, axis=1)
        d = cbn[:, None] - 2.0 * mm
        m = jnp.min(d, axis=0, keepdims=True)
        i = jnp.min(jnp.where(d == m, iota, KC),
                    axis=0, keepdims=True)
        take = m < best
        besti = jnp.where(take, i + c * KC, besti)
        best = jnp.minimum(best, m)
    idx_ref[...] = besti.reshape(1, 1, BT)


_tc_argmin = pl.pallas_call(
    _argmin_body,
    grid=(G,),
    in_specs=[
        pl.BlockSpec((BT, D), lambda i: (i, 0)),
        pl.BlockSpec((K, D), lambda i: (0, 0)),
    ],
    out_specs=pl.BlockSpec((1, 1, BT), lambda i: (i, 0, 0)),
    out_shape=jax.ShapeDtypeStruct((G, 1, BT), jnp.int32),
)


@functools.cache
def _make_sc_gather():
    @functools.partial(
        pl.kernel,
        out_type=jax.ShapeDtypeStruct((TOK, D), jnp.float32),
        mesh=plsc.VectorSubcoreMesh(core_axis_name="c", subcore_axis_name="s"),
        scratch_types=[
            pltpu.VMEM((NCH, CH), jnp.int32),
            pltpu.VMEM((NBUF, CH, D), jnp.float32),
            pltpu.VMEM_SHARED((K, D), jnp.float32),
            pltpu.SemaphoreType.DMA,
            pltpu.SemaphoreType.DMA,
            pltpu.SemaphoreType.DMA,
        ],
    )
    def _sc_gather(table_hbm, idx_hbm, out_hbm, idx_v, buf_v, cb_sh,
                   sem_i, sem_g, sem_w):
        sid = lax.axis_index("s")
        wid = sid * NC + lax.axis_index("c")
        idx_cp = pltpu.async_copy(idx_hbm.at[wid], idx_v, sem_i)

        @pl.when(sid == 0)
        def _():
            pltpu.sync_copy(table_hbm, cb_sh)

        plsc.subcore_barrier()
        idx_cp.wait()

        base = wid * BPW
        gcp = [None] * NCH
        wcp = [None] * NCH
        for j in range(NCH):
            if j >= NBUF:
                wcp[j - NBUF].wait()
            gcp[j] = pltpu.async_copy(cb_sh.at[idx_v.at[j]],
                                      buf_v.at[j % NBUF], sem_g)
            if j >= 1:
                gcp[j - 1].wait()
                wcp[j - 1] = pltpu.async_copy(
                    buf_v.at[(j - 1) % NBUF],
                    out_hbm.at[pl.ds(base + (j - 1) * CH, CH)], sem_w)
        gcp[NCH - 1].wait()
        wcp[NCH - 1] = pltpu.async_copy(
            buf_v.at[(NCH - 1) % NBUF],
            out_hbm.at[pl.ds(base + (NCH - 1) * CH, CH)], sem_w)
        for j in range(NCH - NBUF, NCH):
            wcp[j].wait()

    return _sc_gather


def kernel(x, codebook):
    xf = x.reshape(TOK, D)
    idx = _tc_argmin(xf, codebook).reshape(NW, NCH, CH)
    q = _make_sc_gather()(codebook, idx)
    return q.reshape(B, T, D)
